# Initial kernel scaffold; baseline (speedup 1.0000x reference)
import jax
import jax.numpy as jnp
from jax.experimental import pallas as pl


def kernel(points, index, fc_pos_w, fc_pos_b, blk_fc0_w, blk_fc0_b, blk_fc1_w, blk_fc1_b, blk_sc_w, fc_c_w, fc_c_b, u_e1a_w, u_e1a_b, u_e1b_w, u_e1b_b, u_e2a_w, u_e2a_b, u_e2b_w, u_e2b_b, u_d1a_w, u_d1a_b, u_d1b_w, u_d1b_b, u_out_w, u_out_b):
    raise NotImplementedError("write your pallas kernel here")


# speedup vs baseline: 2.5935x; 2.5935x over previous
import functools

import jax
import jax.numpy as jnp
from jax.experimental import pallas as pl

B, T, DIM = 2, 50000, 3
HID, CD = 128, 32
RESO = 32
S = RESO ** 3
NB = 5
BT = B * T
ROWS = 2000
GRID = BT // ROWS


def _relu(x):
    return jnp.maximum(x, 0.0)


def _dot(a, b):
    return jnp.dot(a, b, preferred_element_type=jnp.float32)


def _head_body(pts_ref, wp_ref, bp_ref, w0_ref, b0_ref, w1_ref, b1_ref,
               ws_ref, out_ref):
    p = pts_ref[...]
    h = _dot(p, wp_ref[...]) + bp_ref[...]
    net = _dot(_relu(h), w0_ref[...]) + b0_ref[...]
    dx = _dot(_relu(net), w1_ref[...]) + b1_ref[...]
    out_ref[...] = _dot(h, ws_ref[...]) + dx


def _block_body(net_ref, pooled_ref, w0_ref, b0_ref, w1_ref, b1_ref,
                ws_ref, out_ref):
    net = net_ref[...]
    pooled = pooled_ref[...]
    w0 = w0_ref[...]
    ws = ws_ref[...]
    h = (_dot(_relu(net), w0[:HID]) + _dot(_relu(pooled), w0[HID:])
         + b0_ref[...])
    dx = _dot(_relu(h), w1_ref[...]) + b1_ref[...]
    out_ref[...] = _dot(net, ws[:HID]) + _dot(pooled, ws[HID:]) + dx


def _block_last_body(net_ref, pooled_ref, w0_ref, b0_ref, w1_ref, b1_ref,
                     ws_ref, wc_ref, bc_ref, out_ref):
    net = net_ref[...]
    pooled = pooled_ref[...]
    w0 = w0_ref[...]
    ws = ws_ref[...]
    h = (_dot(_relu(net), w0[:HID]) + _dot(_relu(pooled), w0[HID:])
         + b0_ref[...])
    dx = _dot(_relu(h), w1_ref[...]) + b1_ref[...]
    out = _dot(net, ws[:HID]) + _dot(pooled, ws[HID:]) + dx
    out_ref[...] = _dot(out, wc_ref[...]) + bc_ref[...]


def _row_spec(cols):
    return pl.BlockSpec((ROWS, cols), lambda i: (i, 0))


def _full_spec(shape):
    nd = len(shape)
    return pl.BlockSpec(shape, lambda i: (0,) * nd)


def _run_head(pts, wp, bp, w0, b0, w1, b1, ws):
    return pl.pallas_call(
        _head_body,
        grid=(GRID,),
        in_specs=[_row_spec(DIM), _full_spec(wp.shape), _full_spec(bp.shape),
                  _full_spec(w0.shape), _full_spec(b0.shape),
                  _full_spec(w1.shape), _full_spec(b1.shape),
                  _full_spec(ws.shape)],
        out_specs=_row_spec(HID),
        out_shape=jax.ShapeDtypeStruct((BT, HID), jnp.float32),
    )(pts, wp, bp, w0, b0, w1, b1, ws)


def _run_block(net, pooled, w0, b0, w1, b1, ws):
    return pl.pallas_call(
        _block_body,
        grid=(GRID,),
        in_specs=[_row_spec(HID), _row_spec(HID),
                  _full_spec(w0.shape), _full_spec(b0.shape),
                  _full_spec(w1.shape), _full_spec(b1.shape),
                  _full_spec(ws.shape)],
        out_specs=_row_spec(HID),
        out_shape=jax.ShapeDtypeStruct((BT, HID), jnp.float32),
    )(net, pooled, w0, b0, w1, b1, ws)


def _run_block_last(net, pooled, w0, b0, w1, b1, ws, wc, bc):
    return pl.pallas_call(
        _block_last_body,
        grid=(GRID,),
        in_specs=[_row_spec(HID), _row_spec(HID),
                  _full_spec(w0.shape), _full_spec(b0.shape),
                  _full_spec(w1.shape), _full_spec(b1.shape),
                  _full_spec(ws.shape), _full_spec(wc.shape),
                  _full_spec(bc.shape)],
        out_specs=_row_spec(CD),
        out_shape=jax.ShapeDtypeStruct((BT, CD), jnp.float32),
    )(net, pooled, w0, b0, w1, b1, ws, wc, bc)


def _pool_local_max(net, idx):
    def one(cb, ib):
        seg = jax.ops.segment_max(cb, ib, num_segments=S)
        seg = jnp.where(jnp.isfinite(seg), seg, 0.0)
        return seg[ib]
    return jax.vmap(one)(net, idx)


def _scatter_mean(c, idx):
    def one(cb, ib):
        s = jax.ops.segment_sum(cb, ib, num_segments=S)
        cnt = jax.ops.segment_sum(jnp.ones((ib.shape[0],), cb.dtype), ib,
                                  num_segments=S)
        return s / jnp.maximum(cnt, 1.0)[:, None]
    return jax.vmap(one)(c, idx)


def _conv3d(x, w, b):
    y = jax.lax.conv_general_dilated(
        x, w, (1, 1, 1), 'SAME', dimension_numbers=('NCDHW', 'OIDHW', 'NCDHW'))
    return y + b[None, :, None, None, None]


def _maxpool(x):
    return jax.lax.reduce_window(x, -jnp.inf, jax.lax.max, (1, 1, 2, 2, 2),
                                 (1, 1, 2, 2, 2), 'VALID')


def _upsample(x):
    x = jnp.repeat(x, 2, axis=2)
    x = jnp.repeat(x, 2, axis=3)
    x = jnp.repeat(x, 2, axis=4)
    return x


def kernel(points, index, fc_pos_w, fc_pos_b, blk_fc0_w, blk_fc0_b,
           blk_fc1_w, blk_fc1_b, blk_sc_w, fc_c_w, fc_c_b,
           u_e1a_w, u_e1a_b, u_e1b_w, u_e1b_b, u_e2a_w, u_e2a_b,
           u_e2b_w, u_e2b_b, u_d1a_w, u_d1a_b, u_d1b_w, u_d1b_b,
           u_out_w, u_out_b):
    idx = index[:, 0, :]
    pts = points.reshape(BT, DIM)
    bp = fc_pos_b.reshape(1, 2 * HID)

    net = _run_head(pts, fc_pos_w, bp, blk_fc0_w[0],
                    blk_fc0_b[0].reshape(1, HID), blk_fc1_w[0],
                    blk_fc1_b[0].reshape(1, HID), blk_sc_w[0])

    for i in range(1, NB - 1):
        pooled = _pool_local_max(net.reshape(B, T, HID), idx).reshape(BT, HID)
        net = _run_block(net, pooled, blk_fc0_w[i],
                         blk_fc0_b[i].reshape(1, HID), blk_fc1_w[i],
                         blk_fc1_b[i].reshape(1, HID), blk_sc_w[i])

    pooled = _pool_local_max(net.reshape(B, T, HID), idx).reshape(BT, HID)
    c = _run_block_last(net, pooled, blk_fc0_w[NB - 1],
                        blk_fc0_b[NB - 1].reshape(1, HID), blk_fc1_w[NB - 1],
                        blk_fc1_b[NB - 1].reshape(1, HID), blk_sc_w[NB - 1],
                        fc_c_w, fc_c_b.reshape(1, CD))

    fea = _scatter_mean(c.reshape(B, T, CD), idx)
    fea = jnp.transpose(fea, (0, 2, 1)).reshape(B, CD, RESO, RESO, RESO)
    e1 = jax.nn.relu(_conv3d(jax.nn.relu(_conv3d(fea, u_e1a_w, u_e1a_b)),
                             u_e1b_w, u_e1b_b))
    p = _maxpool(e1)
    e2 = jax.nn.relu(_conv3d(jax.nn.relu(_conv3d(p, u_e2a_w, u_e2a_b)),
                             u_e2b_w, u_e2b_b))
    u = _upsample(e2)
    d = jnp.concatenate([u, e1], axis=1)
    d = jax.nn.relu(_conv3d(jax.nn.relu(_conv3d(d, u_d1a_w, u_d1a_b)),
                            u_d1b_w, u_d1b_b))
    out = _conv3d(d, u_out_w, u_out_b)
    return out
